# baseline (speedup 1.0000x reference)
import functools

import jax
import jax.numpy as jnp
from jax import lax
from jax.experimental import pallas as pl
from jax.experimental.pallas import tpu as pltpu
from jax.experimental.pallas import tpu_sc as plsc

NSC = 2
NSUB = 16
HF = 64
CK = 128
BR = 512


def _round_up(v, m):
    return (v + m - 1) // m * m


@functools.lru_cache(maxsize=None)
def _make_sc_scatter(np_, ch):
    mesh = plsc.VectorSubcoreMesh(core_axis_name="c", subcore_axis_name="s",
                                  num_cores=NSC, num_subcores=NSUB)
    rpt = np_ // NSUB

    @functools.partial(
        pl.kernel,
        mesh=mesh,
        out_type=jax.ShapeDtypeStruct((NSC, np_, HF), jnp.float32),
        compiler_params=pltpu.CompilerParams(use_tc_tiling_on_sc=False),
        scratch_types=[
            [pltpu.VMEM((2, CK), jnp.int32) for _ in range(2)],
            [pltpu.VMEM((CK, HF), jnp.float32) for _ in range(2)],
            pltpu.VMEM((CK, HF), jnp.float32),
            pltpu.VMEM_SHARED((np_, HF), jnp.float32),
            pltpu.VMEM_SHARED((np_, HF), jnp.float32),
            [pltpu.SemaphoreType.DMA for _ in range(2)],
            [pltpu.SemaphoreType.DMA for _ in range(2)],
            pltpu.SemaphoreType.DMA,
        ],
    )
    def sc_scatter(hs_hbm, rc_hbm, out_hbm,
                   ibufs, bufs, zbuf, table, acc, isems, gsems, stsem):
        c = lax.axis_index("c")
        s = lax.axis_index("s")

        base = s * rpt
        stripe = pl.ds(base, rpt)
        pltpu.async_copy(hs_hbm.at[c, stripe], table.at[stripe], stsem)

        zeros16 = jnp.zeros((16,), jnp.float32)
        nsl = HF // 16

        def _fill(i, _):
            zbuf[i // nsl, pl.ds((i % nsl) * 16, 16)] = zeros16
            return 0

        lax.fori_loop(0, CK * nsl, _fill, 0)

        for t in range(rpt // CK):
            pltpu.sync_copy(zbuf, acc.at[pl.ds(base + t * CK, CK)])
        pltpu.make_async_copy(hs_hbm.at[c, stripe], table.at[stripe],
                              stsem).wait()
        plsc.subcore_barrier()

        edges = rc_hbm.at[s]

        hbm_table = hs_hbm.at[c]
        tables = (table, hbm_table)

        pltpu.async_copy(edges.at[0], ibufs[0], isems[0])
        pltpu.async_copy(edges.at[1], ibufs[1], isems[1])
        pltpu.make_async_copy(edges.at[0], ibufs[0], isems[0]).wait()
        pltpu.async_copy(tables[0].at[ibufs[0].at[0]], bufs[0], gsems[0])

        def _step(j2, _):
            j = j2 * 2
            for p in range(2):
                jj = j + p
                q = 1 - p

                @pl.when(jj + 1 < ch)
                def _():
                    pltpu.make_async_copy(edges.at[jj + 1], ibufs[q],
                                          isems[q]).wait()
                    pltpu.async_copy(tables[q].at[ibufs[q].at[0]], bufs[q],
                                     gsems[q])

                pltpu.make_async_copy(tables[p].at[ibufs[p].at[0]], bufs[p],
                                      gsems[p]).wait()
                pltpu.sync_copy(bufs[p], acc.at[ibufs[p].at[1]], add=True)

                @pl.when(jj + 2 < ch)
                def _():
                    pltpu.async_copy(edges.at[jj + 2], ibufs[p], isems[p])
            return 0

        lax.fori_loop(0, ch // 2, _step, 0)
        plsc.subcore_barrier()

        for t in range(rpt // CK):
            sl = pl.ds(base + t * CK, CK)
            pltpu.sync_copy(acc.at[sl], out_hbm.at[c, sl])

    return sc_scatter


@functools.lru_cache(maxsize=None)
def _make_sc_degree(np_, ch):
    mesh = plsc.VectorSubcoreMesh(core_axis_name="c", subcore_axis_name="s",
                                  num_cores=NSC, num_subcores=NSUB)
    rpt = np_ // NSUB

    @functools.partial(
        pl.kernel,
        mesh=mesh,
        out_type=jax.ShapeDtypeStruct((NSC, np_, 16), jnp.float32),
        compiler_params=pltpu.CompilerParams(use_tc_tiling_on_sc=False),
        scratch_types=[
            pltpu.VMEM((ch, CK), jnp.int32),
            pltpu.VMEM((CK, 16), jnp.float32),
            pltpu.VMEM((CK, 16), jnp.float32),
            pltpu.VMEM_SHARED((np_, 16), jnp.float32),
        ],
    )
    def sc_degree(cols_hbm, out_hbm, col_v, ones_v, zbuf, acc):
        c = lax.axis_index("c")
        s = lax.axis_index("s")

        pltpu.sync_copy(cols_hbm.at[s], col_v)

        ones16 = jnp.ones((16,), jnp.float32)
        zeros16 = jnp.zeros((16,), jnp.float32)

        def _fill(i, _):
            ones_v[i, :] = ones16
            zbuf[i, :] = zeros16
            return 0

        lax.fori_loop(0, CK, _fill, 0)

        base = s * rpt
        for t in range(rpt // CK):
            pltpu.sync_copy(zbuf, acc.at[pl.ds(base + t * CK, CK)])
        plsc.subcore_barrier()

        def _step(j, _):
            pltpu.sync_copy(ones_v, acc.at[col_v.at[j]], add=True)
            return 0

        lax.fori_loop(0, ch, _step, 0)
        plsc.subcore_barrier()

        for t in range(rpt // CK):
            sl = pl.ds(base + t * CK, CK)
            pltpu.sync_copy(acc.at[sl], out_hbm.at[c, sl])

    return sc_degree


def _deg_stats(d0_ref):
    deg = d0_ref[:, 0:1] + 1.0
    dis = lax.rsqrt(deg)
    return deg, dis


def _row_mask(n):
    rid = pl.program_id(0) * BR + lax.broadcasted_iota(jnp.int32, (BR, 1), 0)
    return rid < n


def _split(o_ref, hs):
    o_ref[0] = hs[:, :HF]
    o_ref[1] = hs[:, HF:]


def _unsplit(ref2):
    v = ref2[...]
    return jnp.concatenate([v[0], v[1]], axis=-1)


def _prep1_body(n, x_ref, imp_ref, w_ref, b_ref, d0_ref, o_ref):
    xb = x_ref[...] * imp_ref[...]
    h = jnp.dot(xb, w_ref[...], preferred_element_type=jnp.float32) + b_ref[...]
    _, dis = _deg_stats(d0_ref)
    _split(o_ref, jnp.where(_row_mask(n), dis * h, 0.0))


def _mid_body(n, a_ref, hs_ref, d0_ref, bias_ref, w_ref, b_ref, o_ref):
    deg, dis = _deg_stats(d0_ref)
    sv = _unsplit(a_ref) + _unsplit(hs_ref)
    xk = jnp.maximum(sv * (dis / deg) + bias_ref[...], 0.0)
    h = jnp.dot(xk, w_ref[...], preferred_element_type=jnp.float32) + b_ref[...]
    _split(o_ref, jnp.where(_row_mask(n), dis * h, 0.0))


def _final_body(nclass, a_ref, hs_ref, d0_ref, bias_ref,
                w1_ref, b1_ref, w2_ref, b2_ref, o_ref):
    deg, dis = _deg_stats(d0_ref)
    sv = _unsplit(a_ref) + _unsplit(hs_ref)
    x4 = jnp.maximum(sv * (dis / deg) + bias_ref[...], 0.0)
    h = jnp.maximum(
        jnp.dot(x4, w1_ref[...], preferred_element_type=jnp.float32)
        + b1_ref[...], 0.0)
    o = jnp.dot(h, w2_ref[...], preferred_element_type=jnp.float32) + b2_ref[...]
    cm = lax.broadcasted_iota(jnp.int32, (BR, 128), 1) < nclass
    m = jnp.max(jnp.where(cm, o, -1e30), axis=1, keepdims=True)
    e = jnp.where(cm, jnp.exp(o - m), 0.0)
    o_ref[...] = o - m - jnp.log(jnp.sum(e, axis=1, keepdims=True))


def _vec_spec():
    return pl.BlockSpec((1, 128), lambda i: (0, 0))


def _mat_spec():
    return pl.BlockSpec((128, 128), lambda i: (0, 0))


def _blk_spec():
    return pl.BlockSpec((BR, 128), lambda i: (i, 0))


def _split_spec():
    return pl.BlockSpec((NSC, BR, HF), lambda i: (0, i, 0))


def _deg_spec():
    return pl.BlockSpec((BR, 16), lambda i: (i, 0))


def kernel(x, edge_index, importance, conv1_W, conv1_b, conv1_bias,
           conv2_W, conv2_b, conv2_bias, conv3_W, conv3_b, conv3_bias,
           lin1_W, lin1_b, lin2_W, lin2_b):
    n, d = x.shape
    nclass = lin2_W.shape[0]
    e = edge_index.shape[1]
    np_ = _round_up(n, NSUB * CK)
    ch = max(2, _round_up(-(-e // (NSUB * CK)), 2))
    ep = NSUB * ch * CK
    grid = np_ // BR

    rows = edge_index[0].astype(jnp.int32)
    cols = edge_index[1].astype(jnp.int32)
    pad = jnp.full((ep - e,), n, jnp.int32)
    rows3 = jnp.concatenate([rows, pad]).reshape(NSUB, ch, CK)
    cols3 = jnp.concatenate([cols, pad]).reshape(NSUB, ch, CK)
    rc3 = jnp.stack([rows3, cols3], axis=2)

    xp = jnp.pad(x, ((0, np_ - n), (0, 0)))
    imp = importance.reshape(1, d)
    w1t, w2t, w3t = conv1_W.T, conv2_W.T, conv3_W.T
    l1t = lin1_W.T
    l2t = jnp.pad(lin2_W.T, ((0, 0), (0, 128 - nclass)))
    b1r, b2r, b3r = (conv1_b.reshape(1, d), conv2_b.reshape(1, d),
                     conv3_b.reshape(1, d))
    bias1, bias2, bias3 = (conv1_bias.reshape(1, d), conv2_bias.reshape(1, d),
                           conv3_bias.reshape(1, d))
    l1b = lin1_b.reshape(1, d)
    l2b = jnp.pad(lin2_b.reshape(1, nclass), ((0, 0), (0, 128 - nclass)))

    sc_deg = _make_sc_degree(np_, ch)
    sc_scatter = _make_sc_scatter(np_, ch)

    d0 = sc_deg(cols3)[0]

    split_out = jax.ShapeDtypeStruct((NSC, np_, HF), jnp.float32)

    hs1 = pl.pallas_call(
        functools.partial(_prep1_body, n),
        grid=(grid,),
        in_specs=[_blk_spec(), _vec_spec(), _mat_spec(), _vec_spec(),
                  _deg_spec()],
        out_specs=_split_spec(),
        out_shape=split_out,
    )(xp, imp, w1t, b1r, d0)

    mid = pl.pallas_call(
        functools.partial(_mid_body, n),
        grid=(grid,),
        in_specs=[_split_spec(), _split_spec(), _deg_spec(), _vec_spec(),
                  _mat_spec(), _vec_spec()],
        out_specs=_split_spec(),
        out_shape=split_out,
    )

    agg1 = sc_scatter(hs1, rc3)
    hs2 = mid(agg1, hs1, d0, bias1, w2t, b2r)

    agg2 = sc_scatter(hs2, rc3)
    hs3 = mid(agg2, hs2, d0, bias2, w3t, b3r)

    agg3 = sc_scatter(hs3, rc3)
    out = pl.pallas_call(
        functools.partial(_final_body, nclass),
        grid=(grid,),
        in_specs=[_split_spec(), _split_spec(), _deg_spec(), _vec_spec(),
                  _mat_spec(), _vec_spec(), _mat_spec(), _vec_spec()],
        out_specs=_blk_spec(),
        out_shape=jax.ShapeDtypeStruct((np_, 128), jnp.float32),
    )(agg3, hs3, d0, bias3, l1t, l1b, l2t, l2b)
    return out[:n, :nclass]


# speedup vs baseline: 1.0683x; 1.0683x over previous
import functools

import jax
import jax.numpy as jnp
from jax import lax
from jax.experimental import pallas as pl
from jax.experimental.pallas import tpu as pltpu
from jax.experimental.pallas import tpu_sc as plsc

NSC = 2
NSUB = 16
HF = 64
CK = 128
BR = 512


def _round_up(v, m):
    return (v + m - 1) // m * m


@functools.lru_cache(maxsize=None)
def _make_sc_scatter(np_, ch):
    mesh = plsc.VectorSubcoreMesh(core_axis_name="c", subcore_axis_name="s",
                                  num_cores=NSC, num_subcores=NSUB)
    rpt = np_ // NSUB

    @functools.partial(
        pl.kernel,
        mesh=mesh,
        out_type=jax.ShapeDtypeStruct((NSC, np_, HF), jnp.float32),
        compiler_params=pltpu.CompilerParams(use_tc_tiling_on_sc=False),
        scratch_types=[
            [pltpu.VMEM((2, CK), jnp.int32) for _ in range(2)],
            [pltpu.VMEM((CK, HF), jnp.float32) for _ in range(2)],
            pltpu.VMEM((CK, HF), jnp.float32),
            pltpu.VMEM_SHARED((np_, HF), jnp.float32),
            pltpu.VMEM_SHARED((np_, HF), jnp.float32),
            [pltpu.SemaphoreType.DMA for _ in range(2)],
            [pltpu.SemaphoreType.DMA for _ in range(2)],
            pltpu.SemaphoreType.DMA,
        ],
    )
    def sc_scatter(hs_hbm, rc_hbm, out_hbm,
                   ibufs, bufs, zbuf, table, acc, isems, gsems, stsem):
        c = lax.axis_index("c")
        s = lax.axis_index("s")

        base = s * rpt
        stripe = pl.ds(base, rpt)
        pltpu.async_copy(hs_hbm.at[c, stripe], table.at[stripe], stsem)

        zeros16 = jnp.zeros((16,), jnp.float32)
        nsl = HF // 16

        def _fill(i, _):
            zbuf[i // nsl, pl.ds((i % nsl) * 16, 16)] = zeros16
            return 0

        lax.fori_loop(0, CK * nsl, _fill, 0)

        for t in range(rpt // CK):
            pltpu.sync_copy(zbuf, acc.at[pl.ds(base + t * CK, CK)])
        pltpu.make_async_copy(hs_hbm.at[c, stripe], table.at[stripe],
                              stsem).wait()
        plsc.subcore_barrier()

        edges = rc_hbm.at[s]

        tables = (table, table)

        pltpu.async_copy(edges.at[0], ibufs[0], isems[0])
        pltpu.async_copy(edges.at[1], ibufs[1], isems[1])
        pltpu.make_async_copy(edges.at[0], ibufs[0], isems[0]).wait()
        pltpu.async_copy(tables[0].at[ibufs[0].at[0]], bufs[0], gsems[0])

        def _step(j2, _):
            j = j2 * 2
            for p in range(2):
                jj = j + p
                q = 1 - p

                @pl.when(jj + 1 < ch)
                def _():
                    pltpu.make_async_copy(edges.at[jj + 1], ibufs[q],
                                          isems[q]).wait()
                    pltpu.async_copy(tables[q].at[ibufs[q].at[0]], bufs[q],
                                     gsems[q])

                pltpu.make_async_copy(tables[p].at[ibufs[p].at[0]], bufs[p],
                                      gsems[p]).wait()
                pltpu.sync_copy(bufs[p], acc.at[ibufs[p].at[1]], add=True)

                @pl.when(jj + 2 < ch)
                def _():
                    pltpu.async_copy(edges.at[jj + 2], ibufs[p], isems[p])
            return 0

        lax.fori_loop(0, ch // 2, _step, 0)
        plsc.subcore_barrier()

        for t in range(rpt // CK):
            sl = pl.ds(base + t * CK, CK)
            pltpu.sync_copy(acc.at[sl], out_hbm.at[c, sl])

    return sc_scatter


@functools.lru_cache(maxsize=None)
def _make_sc_degree(np_, ch):
    mesh = plsc.VectorSubcoreMesh(core_axis_name="c", subcore_axis_name="s",
                                  num_cores=NSC, num_subcores=NSUB)
    rpt = np_ // NSUB

    @functools.partial(
        pl.kernel,
        mesh=mesh,
        out_type=jax.ShapeDtypeStruct((NSC, np_, 16), jnp.float32),
        compiler_params=pltpu.CompilerParams(use_tc_tiling_on_sc=False),
        scratch_types=[
            pltpu.VMEM((ch, CK), jnp.int32),
            pltpu.VMEM((CK, 16), jnp.float32),
            pltpu.VMEM((CK, 16), jnp.float32),
            pltpu.VMEM_SHARED((np_, 16), jnp.float32),
        ],
    )
    def sc_degree(cols_hbm, out_hbm, col_v, ones_v, zbuf, acc):
        c = lax.axis_index("c")
        s = lax.axis_index("s")

        pltpu.sync_copy(cols_hbm.at[s], col_v)

        ones16 = jnp.ones((16,), jnp.float32)
        zeros16 = jnp.zeros((16,), jnp.float32)

        def _fill(i, _):
            ones_v[i, :] = ones16
            zbuf[i, :] = zeros16
            return 0

        lax.fori_loop(0, CK, _fill, 0)

        base = s * rpt
        for t in range(rpt // CK):
            pltpu.sync_copy(zbuf, acc.at[pl.ds(base + t * CK, CK)])
        plsc.subcore_barrier()

        def _step(j, _):
            pltpu.sync_copy(ones_v, acc.at[col_v.at[j]], add=True)
            return 0

        lax.fori_loop(0, ch, _step, 0)
        plsc.subcore_barrier()

        for t in range(rpt // CK):
            sl = pl.ds(base + t * CK, CK)
            pltpu.sync_copy(acc.at[sl], out_hbm.at[c, sl])

    return sc_degree


def _deg_stats(d0_ref):
    deg = d0_ref[:, 0:1] + 1.0
    dis = lax.rsqrt(deg)
    return deg, dis


def _row_mask(n):
    rid = pl.program_id(0) * BR + lax.broadcasted_iota(jnp.int32, (BR, 1), 0)
    return rid < n


def _split(o_ref, hs):
    o_ref[0] = hs[:, :HF]
    o_ref[1] = hs[:, HF:]


def _unsplit(ref2):
    v = ref2[...]
    return jnp.concatenate([v[0], v[1]], axis=-1)


def _prep1_body(n, x_ref, imp_ref, w_ref, b_ref, d0_ref, o_ref):
    xb = x_ref[...] * imp_ref[...]
    h = jnp.dot(xb, w_ref[...], preferred_element_type=jnp.float32) + b_ref[...]
    _, dis = _deg_stats(d0_ref)
    _split(o_ref, jnp.where(_row_mask(n), dis * h, 0.0))


def _mid_body(n, a_ref, hs_ref, d0_ref, bias_ref, w_ref, b_ref, o_ref):
    deg, dis = _deg_stats(d0_ref)
    sv = _unsplit(a_ref) + _unsplit(hs_ref)
    xk = jnp.maximum(sv * (dis / deg) + bias_ref[...], 0.0)
    h = jnp.dot(xk, w_ref[...], preferred_element_type=jnp.float32) + b_ref[...]
    _split(o_ref, jnp.where(_row_mask(n), dis * h, 0.0))


def _final_body(nclass, a_ref, hs_ref, d0_ref, bias_ref,
                w1_ref, b1_ref, w2_ref, b2_ref, o_ref):
    deg, dis = _deg_stats(d0_ref)
    sv = _unsplit(a_ref) + _unsplit(hs_ref)
    x4 = jnp.maximum(sv * (dis / deg) + bias_ref[...], 0.0)
    h = jnp.maximum(
        jnp.dot(x4, w1_ref[...], preferred_element_type=jnp.float32)
        + b1_ref[...], 0.0)
    o = jnp.dot(h, w2_ref[...], preferred_element_type=jnp.float32) + b2_ref[...]
    cm = lax.broadcasted_iota(jnp.int32, (BR, 128), 1) < nclass
    m = jnp.max(jnp.where(cm, o, -1e30), axis=1, keepdims=True)
    e = jnp.where(cm, jnp.exp(o - m), 0.0)
    o_ref[...] = o - m - jnp.log(jnp.sum(e, axis=1, keepdims=True))


def _vec_spec():
    return pl.BlockSpec((1, 128), lambda i: (0, 0))


def _mat_spec():
    return pl.BlockSpec((128, 128), lambda i: (0, 0))


def _blk_spec():
    return pl.BlockSpec((BR, 128), lambda i: (i, 0))


def _split_spec():
    return pl.BlockSpec((NSC, BR, HF), lambda i: (0, i, 0))


def _deg_spec():
    return pl.BlockSpec((BR, 16), lambda i: (i, 0))


def kernel(x, edge_index, importance, conv1_W, conv1_b, conv1_bias,
           conv2_W, conv2_b, conv2_bias, conv3_W, conv3_b, conv3_bias,
           lin1_W, lin1_b, lin2_W, lin2_b):
    n, d = x.shape
    nclass = lin2_W.shape[0]
    e = edge_index.shape[1]
    np_ = _round_up(n, NSUB * CK)
    ch = max(2, _round_up(-(-e // (NSUB * CK)), 2))
    ep = NSUB * ch * CK
    grid = np_ // BR

    rows = edge_index[0].astype(jnp.int32)
    cols = edge_index[1].astype(jnp.int32)
    pad = jnp.full((ep - e,), n, jnp.int32)
    rows3 = jnp.concatenate([rows, pad]).reshape(NSUB, ch, CK)
    cols3 = jnp.concatenate([cols, pad]).reshape(NSUB, ch, CK)
    rc3 = jnp.stack([rows3, cols3], axis=2)

    xp = jnp.pad(x, ((0, np_ - n), (0, 0)))
    imp = importance.reshape(1, d)
    w1t, w2t, w3t = conv1_W.T, conv2_W.T, conv3_W.T
    l1t = lin1_W.T
    l2t = jnp.pad(lin2_W.T, ((0, 0), (0, 128 - nclass)))
    b1r, b2r, b3r = (conv1_b.reshape(1, d), conv2_b.reshape(1, d),
                     conv3_b.reshape(1, d))
    bias1, bias2, bias3 = (conv1_bias.reshape(1, d), conv2_bias.reshape(1, d),
                           conv3_bias.reshape(1, d))
    l1b = lin1_b.reshape(1, d)
    l2b = jnp.pad(lin2_b.reshape(1, nclass), ((0, 0), (0, 128 - nclass)))

    sc_deg = _make_sc_degree(np_, ch)
    sc_scatter = _make_sc_scatter(np_, ch)

    d0 = sc_deg(cols3)[0]

    split_out = jax.ShapeDtypeStruct((NSC, np_, HF), jnp.float32)

    hs1 = pl.pallas_call(
        functools.partial(_prep1_body, n),
        grid=(grid,),
        in_specs=[_blk_spec(), _vec_spec(), _mat_spec(), _vec_spec(),
                  _deg_spec()],
        out_specs=_split_spec(),
        out_shape=split_out,
    )(xp, imp, w1t, b1r, d0)

    mid = pl.pallas_call(
        functools.partial(_mid_body, n),
        grid=(grid,),
        in_specs=[_split_spec(), _split_spec(), _deg_spec(), _vec_spec(),
                  _mat_spec(), _vec_spec()],
        out_specs=_split_spec(),
        out_shape=split_out,
    )

    agg1 = sc_scatter(hs1, rc3)
    hs2 = mid(agg1, hs1, d0, bias1, w2t, b2r)

    agg2 = sc_scatter(hs2, rc3)
    hs3 = mid(agg2, hs2, d0, bias2, w3t, b3r)

    agg3 = sc_scatter(hs3, rc3)
    out = pl.pallas_call(
        functools.partial(_final_body, nclass),
        grid=(grid,),
        in_specs=[_split_spec(), _split_spec(), _deg_spec(), _vec_spec(),
                  _mat_spec(), _vec_spec(), _mat_spec(), _vec_spec()],
        out_specs=_blk_spec(),
        out_shape=jax.ShapeDtypeStruct((np_, 128), jnp.float32),
    )(agg3, hs3, d0, bias3, l1t, l1b, l2t, l2b)
    return out[:n, :nclass]


# speedup vs baseline: 1.3944x; 1.3052x over previous
import functools

import jax
import jax.numpy as jnp
from jax import lax
from jax.experimental import pallas as pl
from jax.experimental.pallas import tpu as pltpu
from jax.experimental.pallas import tpu_sc as plsc

NSC = 2
NSUB = 16
HF = 64
CK = 128
BR = 512


def _round_up(v, m):
    return (v + m - 1) // m * m


@functools.lru_cache(maxsize=None)
def _make_sc_scatter(np_, ch):
    mesh = plsc.VectorSubcoreMesh(core_axis_name="c", subcore_axis_name="s",
                                  num_cores=NSC, num_subcores=NSUB)
    rpt = np_ // NSUB

    @functools.partial(
        pl.kernel,
        mesh=mesh,
        out_type=jax.ShapeDtypeStruct((NSC, np_, HF), jnp.float32),
        compiler_params=pltpu.CompilerParams(use_tc_tiling_on_sc=False),
        scratch_types=[
            [pltpu.VMEM((2, CK), jnp.int32) for _ in range(4)],
            [pltpu.VMEM((CK, HF), jnp.float32) for _ in range(4)],
            pltpu.VMEM((CK, HF), jnp.float32),
            pltpu.VMEM_SHARED((np_, HF), jnp.float32),
            pltpu.VMEM_SHARED((np_, HF), jnp.float32),
            [pltpu.SemaphoreType.DMA for _ in range(4)],
            [pltpu.SemaphoreType.DMA for _ in range(4)],
            [pltpu.SemaphoreType.DMA for _ in range(4)],
            pltpu.SemaphoreType.DMA,
        ],
    )
    def sc_scatter(hs_hbm, rc_hbm, out_hbm,
                   ibufs, bufs, zbuf, table, acc, isems, gsems, ssems, stsem):
        c = lax.axis_index("c")
        s = lax.axis_index("s")

        base = s * rpt
        stripe = pl.ds(base, rpt)
        pltpu.async_copy(hs_hbm.at[c, stripe], table.at[stripe], stsem)

        zeros16 = jnp.zeros((16,), jnp.float32)
        nsl = HF // 16

        def _fill(i, _):
            zbuf[i // nsl, pl.ds((i % nsl) * 16, 16)] = zeros16
            return 0

        lax.fori_loop(0, CK * nsl, _fill, 0)

        for t in range(rpt // CK):
            pltpu.sync_copy(zbuf, acc.at[pl.ds(base + t * CK, CK)])
        pltpu.make_async_copy(hs_hbm.at[c, stripe], table.at[stripe],
                              stsem).wait()
        plsc.subcore_barrier()

        edges = rc_hbm.at[s]

        def _idx(jj, b):
            return pltpu.async_copy(edges.at[jj], ibufs[b], isems[b])

        def _gather(b):
            return pltpu.async_copy(table.at[ibufs[b].at[0]], bufs[b],
                                    gsems[b])

        def _scatter(b):
            return pltpu.async_copy(bufs[b], acc.at[ibufs[b].at[1]],
                                    ssems[b], add=True)

        def _swait(b):
            pltpu.make_async_copy(bufs[b], acc.at[ibufs[b].at[1]],
                                  ssems[b]).wait()

        _idx(0, 0)
        _idx(1, 1)
        pltpu.make_async_copy(edges.at[0], ibufs[0], isems[0]).wait()
        _gather(0)

        def _step(j2, _):
            j = j2 * 4
            for p in range(4):
                jj = j + p
                pltpu.make_async_copy(table.at[ibufs[p].at[0]], bufs[p],
                                      gsems[p]).wait()
                _scatter(p)
                b2 = (p + 2) % 4

                @pl.when(jj + 2 < ch)
                def _():
                    @pl.when(jj - 2 >= 0)
                    def _():
                        _swait(b2)

                    _idx(jj + 2, b2)

                b1 = (p + 1) % 4

                @pl.when(jj + 1 < ch)
                def _():
                    pltpu.make_async_copy(edges.at[jj + 1], ibufs[b1],
                                          isems[b1]).wait()
                    _gather(b1)
            return 0

        lax.fori_loop(0, ch // 4, _step, 0)
        for k in range(4):
            _swait((ch - 4 + k) % 4)
        plsc.subcore_barrier()

        for t in range(rpt // CK):
            sl = pl.ds(base + t * CK, CK)
            pltpu.sync_copy(acc.at[sl], out_hbm.at[c, sl])

    return sc_scatter


@functools.lru_cache(maxsize=None)
def _make_sc_degree(np_, ch):
    mesh = plsc.VectorSubcoreMesh(core_axis_name="c", subcore_axis_name="s",
                                  num_cores=NSC, num_subcores=NSUB)
    rpt = np_ // NSUB

    @functools.partial(
        pl.kernel,
        mesh=mesh,
        out_type=jax.ShapeDtypeStruct((NSC, np_, 16), jnp.float32),
        compiler_params=pltpu.CompilerParams(use_tc_tiling_on_sc=False),
        scratch_types=[
            pltpu.VMEM((ch, CK), jnp.int32),
            pltpu.VMEM((CK, 16), jnp.float32),
            pltpu.VMEM((CK, 16), jnp.float32),
            pltpu.VMEM_SHARED((np_, 16), jnp.float32),
        ],
    )
    def sc_degree(cols_hbm, out_hbm, col_v, ones_v, zbuf, acc):
        c = lax.axis_index("c")
        s = lax.axis_index("s")

        pltpu.sync_copy(cols_hbm.at[s], col_v)

        ones16 = jnp.ones((16,), jnp.float32)
        zeros16 = jnp.zeros((16,), jnp.float32)

        def _fill(i, _):
            ones_v[i, :] = ones16
            zbuf[i, :] = zeros16
            return 0

        lax.fori_loop(0, CK, _fill, 0)

        base = s * rpt
        for t in range(rpt // CK):
            pltpu.sync_copy(zbuf, acc.at[pl.ds(base + t * CK, CK)])
        plsc.subcore_barrier()

        def _step(j, _):
            pltpu.sync_copy(ones_v, acc.at[col_v.at[j]], add=True)
            return 0

        lax.fori_loop(0, ch, _step, 0)
        plsc.subcore_barrier()

        for t in range(rpt // CK):
            sl = pl.ds(base + t * CK, CK)
            pltpu.sync_copy(acc.at[sl], out_hbm.at[c, sl])

    return sc_degree


def _deg_stats(d0_ref):
    deg = d0_ref[:, 0:1] + 1.0
    dis = lax.rsqrt(deg)
    return deg, dis


def _row_mask(n):
    rid = pl.program_id(0) * BR + lax.broadcasted_iota(jnp.int32, (BR, 1), 0)
    return rid < n


def _split(o_ref, hs):
    o_ref[0] = hs[:, :HF]
    o_ref[1] = hs[:, HF:]


def _unsplit(ref2):
    v = ref2[...]
    return jnp.concatenate([v[0], v[1]], axis=-1)


def _prep1_body(n, x_ref, imp_ref, w_ref, b_ref, d0_ref, o_ref):
    xb = x_ref[...] * imp_ref[...]
    h = jnp.dot(xb, w_ref[...], preferred_element_type=jnp.float32) + b_ref[...]
    _, dis = _deg_stats(d0_ref)
    _split(o_ref, jnp.where(_row_mask(n), dis * h, 0.0))


def _mid_body(n, a_ref, hs_ref, d0_ref, bias_ref, w_ref, b_ref, o_ref):
    deg, dis = _deg_stats(d0_ref)
    sv = _unsplit(a_ref) + _unsplit(hs_ref)
    xk = jnp.maximum(sv * (dis / deg) + bias_ref[...], 0.0)
    h = jnp.dot(xk, w_ref[...], preferred_element_type=jnp.float32) + b_ref[...]
    _split(o_ref, jnp.where(_row_mask(n), dis * h, 0.0))


def _final_body(nclass, a_ref, hs_ref, d0_ref, bias_ref,
                w1_ref, b1_ref, w2_ref, b2_ref, o_ref):
    deg, dis = _deg_stats(d0_ref)
    sv = _unsplit(a_ref) + _unsplit(hs_ref)
    x4 = jnp.maximum(sv * (dis / deg) + bias_ref[...], 0.0)
    h = jnp.maximum(
        jnp.dot(x4, w1_ref[...], preferred_element_type=jnp.float32)
        + b1_ref[...], 0.0)
    o = jnp.dot(h, w2_ref[...], preferred_element_type=jnp.float32) + b2_ref[...]
    cm = lax.broadcasted_iota(jnp.int32, (BR, 128), 1) < nclass
    m = jnp.max(jnp.where(cm, o, -1e30), axis=1, keepdims=True)
    e = jnp.where(cm, jnp.exp(o - m), 0.0)
    o_ref[...] = o - m - jnp.log(jnp.sum(e, axis=1, keepdims=True))


def _vec_spec():
    return pl.BlockSpec((1, 128), lambda i: (0, 0))


def _mat_spec():
    return pl.BlockSpec((128, 128), lambda i: (0, 0))


def _blk_spec():
    return pl.BlockSpec((BR, 128), lambda i: (i, 0))


def _split_spec():
    return pl.BlockSpec((NSC, BR, HF), lambda i: (0, i, 0))


def _deg_spec():
    return pl.BlockSpec((BR, 16), lambda i: (i, 0))


def kernel(x, edge_index, importance, conv1_W, conv1_b, conv1_bias,
           conv2_W, conv2_b, conv2_bias, conv3_W, conv3_b, conv3_bias,
           lin1_W, lin1_b, lin2_W, lin2_b):
    n, d = x.shape
    nclass = lin2_W.shape[0]
    e = edge_index.shape[1]
    np_ = _round_up(n, NSUB * CK)
    ch = max(8, _round_up(-(-e // (NSUB * CK)), 4))
    ep = NSUB * ch * CK
    grid = np_ // BR

    rows = edge_index[0].astype(jnp.int32)
    cols = edge_index[1].astype(jnp.int32)
    pad = jnp.full((ep - e,), n, jnp.int32)
    rows3 = jnp.concatenate([rows, pad]).reshape(NSUB, ch, CK)
    cols3 = jnp.concatenate([cols, pad]).reshape(NSUB, ch, CK)
    rc3 = jnp.stack([rows3, cols3], axis=2)

    xp = jnp.pad(x, ((0, np_ - n), (0, 0)))
    imp = importance.reshape(1, d)
    w1t, w2t, w3t = conv1_W.T, conv2_W.T, conv3_W.T
    l1t = lin1_W.T
    l2t = jnp.pad(lin2_W.T, ((0, 0), (0, 128 - nclass)))
    b1r, b2r, b3r = (conv1_b.reshape(1, d), conv2_b.reshape(1, d),
                     conv3_b.reshape(1, d))
    bias1, bias2, bias3 = (conv1_bias.reshape(1, d), conv2_bias.reshape(1, d),
                           conv3_bias.reshape(1, d))
    l1b = lin1_b.reshape(1, d)
    l2b = jnp.pad(lin2_b.reshape(1, nclass), ((0, 0), (0, 128 - nclass)))

    sc_deg = _make_sc_degree(np_, ch)
    sc_scatter = _make_sc_scatter(np_, ch)

    d0 = sc_deg(cols3)[0]

    split_out = jax.ShapeDtypeStruct((NSC, np_, HF), jnp.float32)

    hs1 = pl.pallas_call(
        functools.partial(_prep1_body, n),
        grid=(grid,),
        in_specs=[_blk_spec(), _vec_spec(), _mat_spec(), _vec_spec(),
                  _deg_spec()],
        out_specs=_split_spec(),
        out_shape=split_out,
    )(xp, imp, w1t, b1r, d0)

    mid = pl.pallas_call(
        functools.partial(_mid_body, n),
        grid=(grid,),
        in_specs=[_split_spec(), _split_spec(), _deg_spec(), _vec_spec(),
                  _mat_spec(), _vec_spec()],
        out_specs=_split_spec(),
        out_shape=split_out,
    )

    agg1 = sc_scatter(hs1, rc3)
    hs2 = mid(agg1, hs1, d0, bias1, w2t, b2r)

    agg2 = sc_scatter(hs2, rc3)
    hs3 = mid(agg2, hs2, d0, bias2, w3t, b3r)

    agg3 = sc_scatter(hs3, rc3)
    out = pl.pallas_call(
        functools.partial(_final_body, nclass),
        grid=(grid,),
        in_specs=[_split_spec(), _split_spec(), _deg_spec(), _vec_spec(),
                  _mat_spec(), _vec_spec(), _mat_spec(), _vec_spec()],
        out_specs=_blk_spec(),
        out_shape=jax.ShapeDtypeStruct((np_, 128), jnp.float32),
    )(agg3, hs3, d0, bias3, l1t, l1b, l2t, l2b)
    return out[:n, :nclass]


# speedup vs baseline: 1.3952x; 1.0005x over previous
import functools

import jax
import jax.numpy as jnp
from jax import lax
from jax.experimental import pallas as pl
from jax.experimental.pallas import tpu as pltpu
from jax.experimental.pallas import tpu_sc as plsc

NSC = 2
NSUB = 16
HF = 64
CK = 128
BR = 512


def _round_up(v, m):
    return (v + m - 1) // m * m


@functools.lru_cache(maxsize=None)
def _make_sc_scatter(np_, ch):
    mesh = plsc.VectorSubcoreMesh(core_axis_name="c", subcore_axis_name="s",
                                  num_cores=NSC, num_subcores=NSUB)
    rpt = np_ // NSUB

    @functools.partial(
        pl.kernel,
        mesh=mesh,
        out_type=jax.ShapeDtypeStruct((NSC, np_, HF), jnp.float32),
        compiler_params=pltpu.CompilerParams(use_tc_tiling_on_sc=False),
        scratch_types=[
            [pltpu.VMEM((2, CK), jnp.int32) for _ in range(8)],
            [pltpu.VMEM((CK, HF), jnp.float32) for _ in range(4)],
            pltpu.VMEM((CK, HF), jnp.float32),
            pltpu.VMEM_SHARED((np_, HF), jnp.float32),
            pltpu.VMEM_SHARED((np_, HF), jnp.float32),
            [pltpu.SemaphoreType.DMA for _ in range(8)],
            [pltpu.SemaphoreType.DMA for _ in range(4)],
            [pltpu.SemaphoreType.DMA for _ in range(4)],
            pltpu.SemaphoreType.DMA,
        ],
    )
    def sc_scatter(hs_hbm, rc_hbm, out_hbm,
                   ibufs, bufs, zbuf, table, acc, isems, gsems, ssems, stsem):
        c = lax.axis_index("c")
        s = lax.axis_index("s")

        base = s * rpt
        stripe = pl.ds(base, rpt)
        pltpu.async_copy(hs_hbm.at[c, stripe], table.at[stripe], stsem)

        zeros16 = jnp.zeros((16,), jnp.float32)
        nsl = HF // 16

        def _fill(i, _):
            zbuf[i // nsl, pl.ds((i % nsl) * 16, 16)] = zeros16
            return 0

        lax.fori_loop(0, CK * nsl, _fill, 0)

        for t in range(rpt // CK):
            pltpu.sync_copy(zbuf, acc.at[pl.ds(base + t * CK, CK)])
        pltpu.make_async_copy(hs_hbm.at[c, stripe], table.at[stripe],
                              stsem).wait()
        plsc.subcore_barrier()

        edges = rc_hbm.at[s]

        def _idx(jj, b8):
            return pltpu.async_copy(edges.at[jj], ibufs[b8], isems[b8])

        def _gather(b4, b8):
            return pltpu.async_copy(table.at[ibufs[b8].at[0]], bufs[b4],
                                    gsems[b4])

        def _scatter(b4, b8):
            return pltpu.async_copy(bufs[b4], acc.at[ibufs[b8].at[1]],
                                    ssems[b4], add=True)

        def _swait(b4, b8):
            pltpu.make_async_copy(bufs[b4], acc.at[ibufs[b8].at[1]],
                                  ssems[b4]).wait()

        for k in range(4):
            _idx(k, k)
        pltpu.make_async_copy(edges.at[0], ibufs[0], isems[0]).wait()
        _gather(0, 0)

        def _step(j2, _):
            j = j2 * 8
            for p in range(8):
                jj = j + p
                p4 = p % 4
                pltpu.make_async_copy(table.at[ibufs[p].at[0]], bufs[p4],
                                      gsems[p4]).wait()
                _scatter(p4, p)

                @pl.when(jj + 2 < ch)
                def _():
                    @pl.when(jj - 2 >= 0)
                    def _():
                        _swait((p + 2) % 4, (p + 6) % 8)

                @pl.when(jj + 4 < ch)
                def _():
                    _idx(jj + 4, (p + 4) % 8)

                @pl.when(jj + 1 < ch)
                def _():
                    pltpu.make_async_copy(edges.at[jj + 1],
                                          ibufs[(p + 1) % 8],
                                          isems[(p + 1) % 8]).wait()
                    _gather((p + 1) % 4, (p + 1) % 8)
            return 0

        lax.fori_loop(0, ch // 8, _step, 0)
        for k in range(ch - 4, ch):
            _swait(k % 4, k % 8)
        plsc.subcore_barrier()

        for t in range(rpt // CK):
            sl = pl.ds(base + t * CK, CK)
            pltpu.sync_copy(acc.at[sl], out_hbm.at[c, sl])

    return sc_scatter


@functools.lru_cache(maxsize=None)
def _make_sc_degree(np_, ch):
    mesh = plsc.VectorSubcoreMesh(core_axis_name="c", subcore_axis_name="s",
                                  num_cores=NSC, num_subcores=NSUB)
    rpt = np_ // NSUB

    @functools.partial(
        pl.kernel,
        mesh=mesh,
        out_type=jax.ShapeDtypeStruct((NSC, np_, 16), jnp.float32),
        compiler_params=pltpu.CompilerParams(use_tc_tiling_on_sc=False),
        scratch_types=[
            pltpu.VMEM((ch, CK), jnp.int32),
            pltpu.VMEM((CK, 16), jnp.float32),
            pltpu.VMEM((CK, 16), jnp.float32),
            pltpu.VMEM_SHARED((np_, 16), jnp.float32),
        ],
    )
    def sc_degree(cols_hbm, out_hbm, col_v, ones_v, zbuf, acc):
        c = lax.axis_index("c")
        s = lax.axis_index("s")

        pltpu.sync_copy(cols_hbm.at[s], col_v)

        ones16 = jnp.ones((16,), jnp.float32)
        zeros16 = jnp.zeros((16,), jnp.float32)

        def _fill(i, _):
            ones_v[i, :] = ones16
            zbuf[i, :] = zeros16
            return 0

        lax.fori_loop(0, CK, _fill, 0)

        base = s * rpt
        for t in range(rpt // CK):
            pltpu.sync_copy(zbuf, acc.at[pl.ds(base + t * CK, CK)])
        plsc.subcore_barrier()

        def _step(j, _):
            pltpu.sync_copy(ones_v, acc.at[col_v.at[j]], add=True)
            return 0

        lax.fori_loop(0, ch, _step, 0)
        plsc.subcore_barrier()

        for t in range(rpt // CK):
            sl = pl.ds(base + t * CK, CK)
            pltpu.sync_copy(acc.at[sl], out_hbm.at[c, sl])

    return sc_degree


def _deg_stats(d0_ref):
    deg = d0_ref[:, 0:1] + 1.0
    dis = lax.rsqrt(deg)
    return deg, dis


def _row_mask(n):
    rid = pl.program_id(0) * BR + lax.broadcasted_iota(jnp.int32, (BR, 1), 0)
    return rid < n


def _split(o_ref, hs):
    o_ref[0] = hs[:, :HF]
    o_ref[1] = hs[:, HF:]


def _unsplit(ref2):
    v = ref2[...]
    return jnp.concatenate([v[0], v[1]], axis=-1)


def _prep1_body(n, x_ref, imp_ref, w_ref, b_ref, d0_ref, o_ref):
    xb = x_ref[...] * imp_ref[...]
    h = jnp.dot(xb, w_ref[...], preferred_element_type=jnp.float32) + b_ref[...]
    _, dis = _deg_stats(d0_ref)
    _split(o_ref, jnp.where(_row_mask(n), dis * h, 0.0))


def _mid_body(n, a_ref, hs_ref, d0_ref, bias_ref, w_ref, b_ref, o_ref):
    deg, dis = _deg_stats(d0_ref)
    sv = _unsplit(a_ref) + _unsplit(hs_ref)
    xk = jnp.maximum(sv * (dis / deg) + bias_ref[...], 0.0)
    h = jnp.dot(xk, w_ref[...], preferred_element_type=jnp.float32) + b_ref[...]
    _split(o_ref, jnp.where(_row_mask(n), dis * h, 0.0))


def _final_body(nclass, a_ref, hs_ref, d0_ref, bias_ref,
                w1_ref, b1_ref, w2_ref, b2_ref, o_ref):
    deg, dis = _deg_stats(d0_ref)
    sv = _unsplit(a_ref) + _unsplit(hs_ref)
    x4 = jnp.maximum(sv * (dis / deg) + bias_ref[...], 0.0)
    h = jnp.maximum(
        jnp.dot(x4, w1_ref[...], preferred_element_type=jnp.float32)
        + b1_ref[...], 0.0)
    o = jnp.dot(h, w2_ref[...], preferred_element_type=jnp.float32) + b2_ref[...]
    cm = lax.broadcasted_iota(jnp.int32, (BR, 128), 1) < nclass
    m = jnp.max(jnp.where(cm, o, -1e30), axis=1, keepdims=True)
    e = jnp.where(cm, jnp.exp(o - m), 0.0)
    o_ref[...] = o - m - jnp.log(jnp.sum(e, axis=1, keepdims=True))


def _vec_spec():
    return pl.BlockSpec((1, 128), lambda i: (0, 0))


def _mat_spec():
    return pl.BlockSpec((128, 128), lambda i: (0, 0))


def _blk_spec():
    return pl.BlockSpec((BR, 128), lambda i: (i, 0))


def _split_spec():
    return pl.BlockSpec((NSC, BR, HF), lambda i: (0, i, 0))


def _deg_spec():
    return pl.BlockSpec((BR, 16), lambda i: (i, 0))


def kernel(x, edge_index, importance, conv1_W, conv1_b, conv1_bias,
           conv2_W, conv2_b, conv2_bias, conv3_W, conv3_b, conv3_bias,
           lin1_W, lin1_b, lin2_W, lin2_b):
    n, d = x.shape
    nclass = lin2_W.shape[0]
    e = edge_index.shape[1]
    np_ = _round_up(n, NSUB * CK)
    ch = max(8, _round_up(-(-e // (NSUB * CK)), 8))
    ep = NSUB * ch * CK
    grid = np_ // BR

    rows = edge_index[0].astype(jnp.int32)
    cols = edge_index[1].astype(jnp.int32)
    pad = jnp.full((ep - e,), n, jnp.int32)
    rows3 = jnp.concatenate([rows, pad]).reshape(NSUB, ch, CK)
    cols3 = jnp.concatenate([cols, pad]).reshape(NSUB, ch, CK)
    rc3 = jnp.stack([rows3, cols3], axis=2)

    xp = jnp.pad(x, ((0, np_ - n), (0, 0)))
    imp = importance.reshape(1, d)
    w1t, w2t, w3t = conv1_W.T, conv2_W.T, conv3_W.T
    l1t = lin1_W.T
    l2t = jnp.pad(lin2_W.T, ((0, 0), (0, 128 - nclass)))
    b1r, b2r, b3r = (conv1_b.reshape(1, d), conv2_b.reshape(1, d),
                     conv3_b.reshape(1, d))
    bias1, bias2, bias3 = (conv1_bias.reshape(1, d), conv2_bias.reshape(1, d),
                           conv3_bias.reshape(1, d))
    l1b = lin1_b.reshape(1, d)
    l2b = jnp.pad(lin2_b.reshape(1, nclass), ((0, 0), (0, 128 - nclass)))

    sc_deg = _make_sc_degree(np_, ch)
    sc_scatter = _make_sc_scatter(np_, ch)

    d0 = sc_deg(cols3)[0]

    split_out = jax.ShapeDtypeStruct((NSC, np_, HF), jnp.float32)

    hs1 = pl.pallas_call(
        functools.partial(_prep1_body, n),
        grid=(grid,),
        in_specs=[_blk_spec(), _vec_spec(), _mat_spec(), _vec_spec(),
                  _deg_spec()],
        out_specs=_split_spec(),
        out_shape=split_out,
    )(xp, imp, w1t, b1r, d0)

    mid = pl.pallas_call(
        functools.partial(_mid_body, n),
        grid=(grid,),
        in_specs=[_split_spec(), _split_spec(), _deg_spec(), _vec_spec(),
                  _mat_spec(), _vec_spec()],
        out_specs=_split_spec(),
        out_shape=split_out,
    )

    agg1 = sc_scatter(hs1, rc3)
    hs2 = mid(agg1, hs1, d0, bias1, w2t, b2r)

    agg2 = sc_scatter(hs2, rc3)
    hs3 = mid(agg2, hs2, d0, bias2, w3t, b3r)

    agg3 = sc_scatter(hs3, rc3)
    out = pl.pallas_call(
        functools.partial(_final_body, nclass),
        grid=(grid,),
        in_specs=[_split_spec(), _split_spec(), _deg_spec(), _vec_spec(),
                  _mat_spec(), _vec_spec(), _mat_spec(), _vec_spec()],
        out_specs=_blk_spec(),
        out_shape=jax.ShapeDtypeStruct((np_, 128), jnp.float32),
    )(agg3, hs3, d0, bias3, l1t, l1b, l2t, l2b)
    return out[:n, :nclass]


# speedup vs baseline: 1.5656x; 1.1221x over previous
import functools

import jax
import jax.numpy as jnp
from jax import lax
from jax.experimental import pallas as pl
from jax.experimental.pallas import tpu as pltpu
from jax.experimental.pallas import tpu_sc as plsc

NSC = 2
NSUB = 16
HF = 64
CK = 128
BR = 512


def _round_up(v, m):
    return (v + m - 1) // m * m


@functools.lru_cache(maxsize=None)
def _make_sc_scatter(np_, ch):
    mesh = plsc.VectorSubcoreMesh(core_axis_name="c", subcore_axis_name="s",
                                  num_cores=NSC, num_subcores=NSUB)
    rpt = np_ // NSUB

    @functools.partial(
        pl.kernel,
        mesh=mesh,
        out_type=jax.ShapeDtypeStruct((np_, 128), jnp.float32),
        compiler_params=pltpu.CompilerParams(use_tc_tiling_on_sc=False),
        scratch_types=[
            [pltpu.VMEM((2, CK), jnp.int32) for _ in range(8)],
            [pltpu.VMEM((CK, HF), jnp.float32) for _ in range(4)],
            pltpu.VMEM((CK, HF), jnp.float32),
            pltpu.VMEM_SHARED((np_, HF), jnp.float32),
            pltpu.VMEM_SHARED((np_, HF), jnp.float32),
            [pltpu.SemaphoreType.DMA for _ in range(8)],
            [pltpu.SemaphoreType.DMA for _ in range(4)],
            [pltpu.SemaphoreType.DMA for _ in range(4)],
            pltpu.SemaphoreType.DMA,
        ],
    )
    def sc_scatter(hs_hbm, rc_hbm, out_hbm,
                   ibufs, bufs, zbuf, table, acc, isems, gsems, ssems, stsem):
        c = lax.axis_index("c")
        s = lax.axis_index("s")

        base = s * rpt
        stripe = pl.ds(base, rpt)
        fcols = pl.ds(c * HF, HF)
        pltpu.async_copy(hs_hbm.at[stripe, fcols], table.at[stripe], stsem)

        zeros16 = jnp.zeros((16,), jnp.float32)
        nsl = HF // 16

        def _fill(i, _):
            zbuf[i // nsl, pl.ds((i % nsl) * 16, 16)] = zeros16
            return 0

        lax.fori_loop(0, CK * nsl, _fill, 0)

        for t in range(rpt // CK):
            pltpu.sync_copy(zbuf, acc.at[pl.ds(base + t * CK, CK)])
        pltpu.make_async_copy(hs_hbm.at[stripe, fcols], table.at[stripe],
                              stsem).wait()
        plsc.subcore_barrier()

        edges = rc_hbm.at[s]

        def _idx(jj, b8):
            return pltpu.async_copy(edges.at[jj], ibufs[b8], isems[b8])

        def _gather(b4, b8):
            return pltpu.async_copy(table.at[ibufs[b8].at[0]], bufs[b4],
                                    gsems[b4])

        def _scatter(b4, b8):
            return pltpu.async_copy(bufs[b4], acc.at[ibufs[b8].at[1]],
                                    ssems[b4], add=True)

        def _swait(b4, b8):
            pltpu.make_async_copy(bufs[b4], acc.at[ibufs[b8].at[1]],
                                  ssems[b4]).wait()

        for k in range(4):
            _idx(k, k)
        pltpu.make_async_copy(edges.at[0], ibufs[0], isems[0]).wait()
        _gather(0, 0)

        def _step(j2, _):
            j = j2 * 8
            for p in range(8):
                jj = j + p
                p4 = p % 4
                pltpu.make_async_copy(table.at[ibufs[p].at[0]], bufs[p4],
                                      gsems[p4]).wait()
                _scatter(p4, p)

                @pl.when(jj + 2 < ch)
                def _():
                    @pl.when(jj - 2 >= 0)
                    def _():
                        _swait((p + 2) % 4, (p + 6) % 8)

                @pl.when(jj + 4 < ch)
                def _():
                    _idx(jj + 4, (p + 4) % 8)

                @pl.when(jj + 1 < ch)
                def _():
                    pltpu.make_async_copy(edges.at[jj + 1],
                                          ibufs[(p + 1) % 8],
                                          isems[(p + 1) % 8]).wait()
                    _gather((p + 1) % 4, (p + 1) % 8)
            return 0

        lax.fori_loop(0, ch // 8, _step, 0)
        for k in range(ch - 4, ch):
            _swait(k % 4, k % 8)
        plsc.subcore_barrier()

        for t in range(rpt // CK):
            sl = pl.ds(base + t * CK, CK)
            pltpu.sync_copy(acc.at[sl], out_hbm.at[sl, fcols])

    return sc_scatter


@functools.lru_cache(maxsize=None)
def _make_sc_degree(np_, ch):
    mesh = plsc.VectorSubcoreMesh(core_axis_name="c", subcore_axis_name="s",
                                  num_cores=NSC, num_subcores=NSUB)
    rpt = np_ // NSUB

    @functools.partial(
        pl.kernel,
        mesh=mesh,
        out_type=jax.ShapeDtypeStruct((NSC, np_, 16), jnp.float32),
        compiler_params=pltpu.CompilerParams(use_tc_tiling_on_sc=False),
        scratch_types=[
            pltpu.VMEM((ch, CK), jnp.int32),
            pltpu.VMEM((CK, 16), jnp.float32),
            pltpu.VMEM((CK, 16), jnp.float32),
            pltpu.VMEM_SHARED((np_, 16), jnp.float32),
        ],
    )
    def sc_degree(cols_hbm, out_hbm, col_v, ones_v, zbuf, acc):
        c = lax.axis_index("c")
        s = lax.axis_index("s")

        pltpu.sync_copy(cols_hbm.at[s], col_v)

        ones16 = jnp.ones((16,), jnp.float32)
        zeros16 = jnp.zeros((16,), jnp.float32)

        def _fill(i, _):
            ones_v[i, :] = ones16
            zbuf[i, :] = zeros16
            return 0

        lax.fori_loop(0, CK, _fill, 0)

        base = s * rpt
        for t in range(rpt // CK):
            pltpu.sync_copy(zbuf, acc.at[pl.ds(base + t * CK, CK)])
        plsc.subcore_barrier()

        def _step(j, _):
            pltpu.sync_copy(ones_v, acc.at[col_v.at[j]], add=True)
            return 0

        lax.fori_loop(0, ch, _step, 0)
        plsc.subcore_barrier()

        for t in range(rpt // CK):
            sl = pl.ds(base + t * CK, CK)
            pltpu.sync_copy(acc.at[sl], out_hbm.at[c, sl])

    return sc_degree


def _deg_stats(d0_ref):
    deg = d0_ref[:, 0:1] + 1.0
    dis = lax.rsqrt(deg)
    return deg, dis


def _row_mask(n):
    rid = pl.program_id(0) * BR + lax.broadcasted_iota(jnp.int32, (BR, 1), 0)
    return rid < n


def _prep1_body(n, x_ref, imp_ref, w_ref, b_ref, d0_ref, o_ref):
    xb = x_ref[...] * imp_ref[...]
    h = jnp.dot(xb, w_ref[...], preferred_element_type=jnp.float32) + b_ref[...]
    _, dis = _deg_stats(d0_ref)
    o_ref[...] = jnp.where(_row_mask(n), dis * h, 0.0)


def _mid_body(n, a_ref, hs_ref, d0_ref, bias_ref, w_ref, b_ref, o_ref):
    deg, dis = _deg_stats(d0_ref)
    sv = a_ref[...] + hs_ref[...]
    xk = jnp.maximum(sv * (dis / deg) + bias_ref[...], 0.0)
    h = jnp.dot(xk, w_ref[...], preferred_element_type=jnp.float32) + b_ref[...]
    o_ref[...] = jnp.where(_row_mask(n), dis * h, 0.0)


def _final_body(nclass, a_ref, hs_ref, d0_ref, bias_ref,
                w1_ref, b1_ref, w2_ref, b2_ref, o_ref):
    deg, dis = _deg_stats(d0_ref)
    sv = a_ref[...] + hs_ref[...]
    x4 = jnp.maximum(sv * (dis / deg) + bias_ref[...], 0.0)
    h = jnp.maximum(
        jnp.dot(x4, w1_ref[...], preferred_element_type=jnp.float32)
        + b1_ref[...], 0.0)
    o = jnp.dot(h, w2_ref[...], preferred_element_type=jnp.float32) + b2_ref[...]
    cm = lax.broadcasted_iota(jnp.int32, (BR, 128), 1) < nclass
    m = jnp.max(jnp.where(cm, o, -1e30), axis=1, keepdims=True)
    e = jnp.where(cm, jnp.exp(o - m), 0.0)
    o_ref[...] = o - m - jnp.log(jnp.sum(e, axis=1, keepdims=True))


def _vec_spec():
    return pl.BlockSpec((1, 128), lambda i: (0, 0))


def _mat_spec():
    return pl.BlockSpec((128, 128), lambda i: (0, 0))


def _blk_spec():
    return pl.BlockSpec((BR, 128), lambda i: (i, 0))


def _deg_spec():
    return pl.BlockSpec((BR, 16), lambda i: (i, 0))


def kernel(x, edge_index, importance, conv1_W, conv1_b, conv1_bias,
           conv2_W, conv2_b, conv2_bias, conv3_W, conv3_b, conv3_bias,
           lin1_W, lin1_b, lin2_W, lin2_b):
    n, d = x.shape
    nclass = lin2_W.shape[0]
    e = edge_index.shape[1]
    np_ = _round_up(n, NSUB * CK)
    ch = max(8, _round_up(-(-e // (NSUB * CK)), 8))
    ep = NSUB * ch * CK
    grid = np_ // BR

    rows = edge_index[0].astype(jnp.int32)
    cols = edge_index[1].astype(jnp.int32)
    pad = jnp.full((ep - e,), n, jnp.int32)
    rows3 = jnp.concatenate([rows, pad]).reshape(NSUB, ch, CK)
    cols3 = jnp.concatenate([cols, pad]).reshape(NSUB, ch, CK)
    rc3 = jnp.stack([rows3, cols3], axis=2)

    xp = jnp.pad(x, ((0, np_ - n), (0, 0)))
    imp = importance.reshape(1, d)
    w1t, w2t, w3t = conv1_W.T, conv2_W.T, conv3_W.T
    l1t = lin1_W.T
    l2t = jnp.pad(lin2_W.T, ((0, 0), (0, 128 - nclass)))
    b1r, b2r, b3r = (conv1_b.reshape(1, d), conv2_b.reshape(1, d),
                     conv3_b.reshape(1, d))
    bias1, bias2, bias3 = (conv1_bias.reshape(1, d), conv2_bias.reshape(1, d),
                           conv3_bias.reshape(1, d))
    l1b = lin1_b.reshape(1, d)
    l2b = jnp.pad(lin2_b.reshape(1, nclass), ((0, 0), (0, 128 - nclass)))

    sc_deg = _make_sc_degree(np_, ch)
    sc_scatter = _make_sc_scatter(np_, ch)

    d0 = sc_deg(cols3)[0]

    wide_out = jax.ShapeDtypeStruct((np_, 128), jnp.float32)

    hs1 = pl.pallas_call(
        functools.partial(_prep1_body, n),
        grid=(grid,),
        in_specs=[_blk_spec(), _vec_spec(), _mat_spec(), _vec_spec(),
                  _deg_spec()],
        out_specs=_blk_spec(),
        out_shape=wide_out,
    )(xp, imp, w1t, b1r, d0)

    mid = pl.pallas_call(
        functools.partial(_mid_body, n),
        grid=(grid,),
        in_specs=[_blk_spec(), _blk_spec(), _deg_spec(), _vec_spec(),
                  _mat_spec(), _vec_spec()],
        out_specs=_blk_spec(),
        out_shape=wide_out,
    )

    agg1 = sc_scatter(hs1, rc3)
    hs2 = mid(agg1, hs1, d0, bias1, w2t, b2r)

    agg2 = sc_scatter(hs2, rc3)
    hs3 = mid(agg2, hs2, d0, bias2, w3t, b3r)

    agg3 = sc_scatter(hs3, rc3)
    out = pl.pallas_call(
        functools.partial(_final_body, nclass),
        grid=(grid,),
        in_specs=[_blk_spec(), _blk_spec(), _deg_spec(), _vec_spec(),
                  _mat_spec(), _vec_spec(), _mat_spec(), _vec_spec()],
        out_specs=_blk_spec(),
        out_shape=wide_out,
    )(agg3, hs3, d0, bias3, l1t, l1b, l2t, l2b)
    return out[:n, :nclass]
